# Initial kernel scaffold; baseline (speedup 1.0000x reference)
import jax
import jax.numpy as jnp
from jax.experimental import pallas as pl


def kernel(user_feat_0, user_feat_1, user_feat_2, user_feat_3, item_feat_0, item_feat_1, item_feat_2, item_feat_3, user_ids, item_ids, adj_indices, adj_values, user_emb, item_emb, Wu0, Wu1, Wu2, Wu3, Wi0, Wi1, Wi2, Wi3, fc1_w, fc1_b, fc2_w, fc2_b, out_w, out_b):
    raise NotImplementedError("write your pallas kernel here")


# speedup vs baseline: 8.9557x; 8.9557x over previous
import functools

import jax
import jax.numpy as jnp
from jax import lax
from jax.experimental import pallas as pl
from jax.experimental.pallas import tpu as pltpu
from jax.experimental.pallas import tpu_sc as plsc

NUM_USERS = 50000
NUM_ITEMS = 50000
N_NODES = NUM_USERS + NUM_ITEMS
HALF = N_NODES // 2
EMB = 32
N_EDGES = 1600000
CHUNK = 128
N_CHUNKS = N_EDGES // CHUNK
BLK = 4
BASE_CHUNKS = N_CHUNKS // 16
FULL_BLOCKS = BASE_CHUNKS // BLK
EXTRA_TILES = N_CHUNKS - 16 * BASE_CHUNKS
DUMMY_ROWS = BLK * CHUNK
ACC_ROWS = HALF + DUMMY_ROWS
BASE_G = 390
EXTRA_G_TILES = 10

_mesh = plsc.VectorSubcoreMesh(core_axis_name="c", subcore_axis_name="s")


@functools.partial(
    pl.kernel,
    out_type=jax.ShapeDtypeStruct((N_NODES, EMB), jnp.float32),
    mesh=_mesh,
    compiler_params=pltpu.CompilerParams(use_tc_tiling_on_sc=False),
    scratch_types=[
        pltpu.VMEM((BLK * CHUNK,), jnp.int32),
        pltpu.VMEM((BLK * CHUNK,), jnp.int32),
        pltpu.VMEM((BLK * CHUNK,), jnp.float32),
        pltpu.VMEM((BLK, CHUNK), jnp.int32),
        pltpu.VMEM((BLK, CHUNK, EMB), jnp.float32),
        pltpu.VMEM_SHARED((ACC_ROWS, EMB), jnp.float32),
        pltpu.SemaphoreType.DMA,
    ],
)
def _spmm_layer(emb_hbm, src_hbm, dst_hbm, val_hbm, out_hbm,
                src_v, dst_v, val_v, sidx_v, rows_v, acc, sem):
    c = lax.axis_index("c")
    s = lax.axis_index("s")
    base_node = c * HALF
    iota16 = lax.iota(jnp.int32, 16)

    def zz(i, _):
        for j in range(BLK):
            rows_v[j, i, pl.ds(0, 16)] = jnp.zeros((16,), jnp.float32)
            rows_v[j, i, pl.ds(16, 16)] = jnp.zeros((16,), jnp.float32)
        return 0
    lax.fori_loop(0, CHUNK, zz, 0)
    row_base = s * (BASE_G * 8) + 8 * jnp.minimum(s, EXTRA_G_TILES)

    def zcopy(k, _):
        pltpu.sync_copy(rows_v.at[0], acc.at[pl.ds(row_base + k * CHUNK, CHUNK)])
        return 0
    lax.fori_loop(0, 24, zcopy, 0)
    pltpu.sync_copy(rows_v.at[0, pl.ds(0, 48)], acc.at[pl.ds(row_base + 3072, 48)])

    @pl.when(s < EXTRA_G_TILES)
    def _():
        pltpu.sync_copy(rows_v.at[0, pl.ds(0, 8)], acc.at[pl.ds(row_base + 3120, 8)])
    plsc.subcore_barrier()

    chunk0 = s * BASE_CHUNKS + jnp.minimum(s, EXTRA_TILES)
    nch = jnp.where(s < EXTRA_TILES, BASE_CHUNKS + 1, BASE_CHUNKS)

    def compute_chunk(j):
        for cc in range(8):
            off = cc * 16
            dstv = dst_v[pl.ds(j * CHUNK + off, 16)] - base_node
            in_range = (dstv >= 0) & (dstv < HALF)
            dummy = HALF + j * CHUNK + off + iota16
            sidx_v[j, pl.ds(off, 16)] = jnp.where(in_range, dstv, dummy)
            valv = val_v[pl.ds(j * CHUNK + off, 16)]
            for e in range(16):
                vs = jnp.broadcast_to(valv[e], (16,))
                ce = off + e
                rows_v[j, ce, pl.ds(0, 16)] = rows_v[j, ce, pl.ds(0, 16)] * vs
                rows_v[j, ce, pl.ds(16, 16)] = rows_v[j, ce, pl.ds(16, 16)] * vs

    def do_block(e0, nj_static):
        ne = nj_static * CHUNK
        pltpu.sync_copy(src_hbm.at[pl.ds(e0, ne)], src_v.at[pl.ds(0, ne)])
        pltpu.sync_copy(dst_hbm.at[pl.ds(e0, ne)], dst_v.at[pl.ds(0, ne)])
        pltpu.sync_copy(val_hbm.at[pl.ds(e0, ne)], val_v.at[pl.ds(0, ne)])
        handles = []
        for j in range(nj_static):
            handles.append(pltpu.async_copy(
                emb_hbm.at[src_v.at[pl.ds(j * CHUNK, CHUNK)]], rows_v.at[j], sem))
        for h in handles:
            h.wait()
        for j in range(nj_static):
            compute_chunk(j)
            pltpu.sync_copy(rows_v.at[j], acc.at[sidx_v.at[j]], add=True)

    def blk_body(b, _):
        do_block((chunk0 + b * BLK) * CHUNK, BLK)
        return 0
    lax.fori_loop(0, FULL_BLOCKS, blk_body, 0)

    def rem_body(r, _):
        do_block((chunk0 + FULL_BLOCKS * BLK + r) * CHUNK, 1)
        return 0
    lax.fori_loop(0, nch - FULL_BLOCKS * BLK, rem_body, 0)

    plsc.subcore_barrier()

    out_base = base_node + row_base
    pltpu.sync_copy(acc.at[pl.ds(row_base, 1024)], out_hbm.at[pl.ds(out_base, 1024)])
    pltpu.sync_copy(acc.at[pl.ds(row_base + 1024, 1024)], out_hbm.at[pl.ds(out_base + 1024, 1024)])
    pltpu.sync_copy(acc.at[pl.ds(row_base + 2048, 1024)], out_hbm.at[pl.ds(out_base + 2048, 1024)])
    pltpu.sync_copy(acc.at[pl.ds(row_base + 3072, 48)], out_hbm.at[pl.ds(out_base + 3072, 48)])

    @pl.when(s < EXTRA_G_TILES)
    def _():
        pltpu.sync_copy(acc.at[pl.ds(row_base + 3120, 8)], out_hbm.at[pl.ds(out_base + 3120, 8)])


@functools.partial(
    pl.kernel,
    out_type=jax.ShapeDtypeStruct((8192, EMB), jnp.float32),
    mesh=_mesh,
    compiler_params=pltpu.CompilerParams(use_tc_tiling_on_sc=False),
    scratch_types=[
        pltpu.VMEM((256,), jnp.int32),
        pltpu.VMEM((2, CHUNK, EMB), jnp.float32),
        pltpu.VMEM((2, CHUNK, EMB), jnp.float32),
        pltpu.VMEM((2, CHUNK, EMB), jnp.float32),
        pltpu.SemaphoreType.DMA,
    ],
)
def _gather_mean(e0_hbm, e1_hbm, e2_hbm, ids_hbm, out_hbm,
                 idx_v, a_v, b_v, c_v, sem):
    c = lax.axis_index("c")
    s = lax.axis_index("s")
    wid = s * 2 + c
    pltpu.sync_copy(ids_hbm.at[pl.ds(wid * 256, 256)], idx_v)
    handles = []
    for j in range(2):
        isl = idx_v.at[pl.ds(j * CHUNK, CHUNK)]
        handles.append(pltpu.async_copy(e0_hbm.at[isl], a_v.at[j], sem))
        handles.append(pltpu.async_copy(e1_hbm.at[isl], b_v.at[j], sem))
        handles.append(pltpu.async_copy(e2_hbm.at[isl], c_v.at[j], sem))
    for h in handles:
        h.wait()
    third = jnp.full((16,), 1.0 / 3.0, jnp.float32)
    for j in range(2):
        def mean_row(r, _):
            for hh in range(2):
                sl = pl.ds(hh * 16, 16)
                a_v[j, r, sl] = (a_v[j, r, sl] + b_v[j, r, sl] + c_v[j, r, sl]) * third
            return 0
        lax.fori_loop(0, CHUNK, mean_row, 0)
        pltpu.sync_copy(a_v.at[j], out_hbm.at[pl.ds(wid * 256 + j * CHUNK, CHUNK)])


def _mlp_body(uf0, uf1, uf2, uf3, if0, if1, if2, if3, ug, ig,
              wu0, wu1, wu2, wu3, wi0, wi1, wi2, wi3,
              fc1w, fc1b, fc2w, fc2b, outw, out_ref):
    def dot_t(a, b):
        return lax.dot_general(a, b, (((1,), (1,)), ((), ())),
                               precision=lax.Precision.HIGHEST,
                               preferred_element_type=jnp.float32)

    h1 = dot_t(ug[...], fc1w[:, 128:160])
    h1 = h1 + dot_t(ig[...], fc1w[:, 288:320])
    ufs = (uf0, uf1, uf2, uf3)
    wus = (wu0, wu1, wu2, wu3)
    ifs = (if0, if1, if2, if3)
    wis = (wi0, wi1, wi2, wi3)
    for f in range(4):
        tu = dot_t(ufs[f][...], wus[f][...])
        h1 = h1 + dot_t(tu, fc1w[:, 32 * f:32 * f + 32])
        ti = dot_t(ifs[f][...], wis[f][...])
        h1 = h1 + dot_t(ti, fc1w[:, 160 + 32 * f:160 + 32 * f + 32])
    h1 = jnp.maximum(h1 + fc1b[...], 0.0)
    h2 = jnp.maximum(dot_t(h1, fc2w[...]) + fc2b[...], 0.0)
    out_ref[...] = dot_t(h2, outw[...])


def kernel(user_feat_0, user_feat_1, user_feat_2, user_feat_3,
           item_feat_0, item_feat_1, item_feat_2, item_feat_3,
           user_ids, item_ids, adj_indices, adj_values,
           user_emb, item_emb,
           Wu0, Wu1, Wu2, Wu3, Wi0, Wi1, Wi2, Wi3,
           fc1_w, fc1_b, fc2_w, fc2_b, out_w, out_b):
    src1d = adj_indices[0].reshape(N_EDGES)
    dst1d = adj_indices[1].reshape(N_EDGES)
    val1d = adj_values.reshape(N_EDGES)

    emb0 = jnp.concatenate([user_emb, item_emb], axis=0)
    emb1 = _spmm_layer(emb0, src1d, dst1d, val1d)
    emb2 = _spmm_layer(emb1, src1d, dst1d, val1d)

    ids1d = jnp.concatenate([user_ids, item_ids + NUM_USERS]).astype(jnp.int32)
    gcn = _gather_mean(emb0, emb1, emb2, ids1d)
    ugcn = gcn[:4096]
    igcn = gcn[4096:]

    bs = 512
    grid = (4096 // bs,)
    feat_spec = pl.BlockSpec((bs, 512), lambda i: (i, 0))
    gcn_spec = pl.BlockSpec((bs, EMB), lambda i: (i, 0))
    w_spec = pl.BlockSpec((EMB, 512), lambda i: (0, 0))
    out = pl.pallas_call(
        _mlp_body,
        grid=grid,
        in_specs=[feat_spec] * 4 + [feat_spec] * 4 + [gcn_spec] * 2
        + [w_spec] * 8
        + [pl.BlockSpec((128, 320), lambda i: (0, 0)),
           pl.BlockSpec((1, 128), lambda i: (0, 0)),
           pl.BlockSpec((64, 128), lambda i: (0, 0)),
           pl.BlockSpec((1, 64), lambda i: (0, 0)),
           pl.BlockSpec((1, 64), lambda i: (0, 0))],
        out_specs=pl.BlockSpec((bs, 1), lambda i: (i, 0)),
        out_shape=jax.ShapeDtypeStruct((4096, 1), jnp.float32),
    )(user_feat_0, user_feat_1, user_feat_2, user_feat_3,
      item_feat_0, item_feat_1, item_feat_2, item_feat_3,
      ugcn, igcn,
      Wu0, Wu1, Wu2, Wu3, Wi0, Wi1, Wi2, Wi3,
      fc1_w, fc1_b.reshape(1, 128), fc2_w, fc2_b.reshape(1, 64),
      out_w)
    return out + out_b


# speedup vs baseline: 19.0552x; 2.1277x over previous
import functools

import jax
import jax.numpy as jnp
from jax import lax
from jax.experimental import pallas as pl
from jax.experimental.pallas import tpu as pltpu
from jax.experimental.pallas import tpu_sc as plsc

NUM_USERS = 50000
NUM_ITEMS = 50000
N_NODES = NUM_USERS + NUM_ITEMS
HALF = N_NODES // 2
EMB = 32
N_EDGES = 1600000
CHUNK = 128
N_CHUNKS = N_EDGES // CHUNK
BLK = 2
DUMMY_ROWS = BLK * CHUNK
ACC_ROWS = HALF + DUMMY_ROWS
SEG_CHUNKS = 396
SEG_CAP = SEG_CHUNKS * CHUNK
N_SEGS = 64
P_BASE_CHUNKS = N_CHUNKS // 32
P_EXTRA = N_CHUNKS - 32 * P_BASE_CHUNKS
BASE_G = 390
EXTRA_G_TILES = 10

_mesh = plsc.VectorSubcoreMesh(core_axis_name="c", subcore_axis_name="s")


@functools.partial(
    pl.kernel,
    out_type=[
        jax.ShapeDtypeStruct((N_SEGS * SEG_CAP,), jnp.int32),
        jax.ShapeDtypeStruct((N_SEGS * SEG_CAP,), jnp.int32),
        jax.ShapeDtypeStruct((N_SEGS * SEG_CAP,), jnp.float32),
        jax.ShapeDtypeStruct((N_SEGS, 16), jnp.int32),
    ],
    mesh=_mesh,
    compiler_params=pltpu.CompilerParams(use_tc_tiling_on_sc=False,
                                         needs_layout_passes=False),
    scratch_types=[
        pltpu.VMEM((2, 1024), jnp.int32),
        pltpu.VMEM((2, 1024), jnp.int32),
        pltpu.VMEM((2, 1024), jnp.float32),
        pltpu.VMEM((2304,), jnp.int32),
        pltpu.VMEM((2304,), jnp.int32),
        pltpu.VMEM((2304,), jnp.int32),
        pltpu.VMEM((2304,), jnp.int32),
        pltpu.VMEM((2304,), jnp.float32),
        pltpu.VMEM((2304,), jnp.float32),
        pltpu.VMEM((CHUNK,), jnp.int32),
        pltpu.VMEM((CHUNK,), jnp.int32),
        pltpu.VMEM((CHUNK,), jnp.float32),
        pltpu.VMEM((16,), jnp.int32),
        pltpu.SemaphoreType.DMA,
    ],
)
def _partition_edges(src_hbm, dst_hbm, val_hbm,
                     osrc, odst, oval, ocnt,
                     isrc, idst, ival, bsrc0, bsrc1, bdst0, bdst1,
                     bval0, bval1, psrc, pdst, pval, cnt_v, sem_in):
    bsrc = (bsrc0, bsrc1)
    bdst = (bdst0, bdst1)
    bval = (bval0, bval1)
    c = lax.axis_index("c")
    s = lax.axis_index("s")
    w = s * 2 + c
    iota16 = lax.iota(jnp.int32, 16)
    chunkA = w * P_BASE_CHUNKS + jnp.minimum(w, P_EXTRA)
    ncw = jnp.where(w < P_EXTRA, P_BASE_CHUNKS + 1, P_BASE_CHUNKS)
    NBI = 48

    def fillpad(i, _):
        sl = pl.ds(i * 16, 16)
        psrc[sl] = w * CHUNK + i * 16 + iota16
        pdst[sl] = jnp.full((16,), HALF, jnp.int32)
        pval[sl] = jnp.zeros((16,), jnp.float32)
        return 0
    lax.fori_loop(0, 8, fillpad, 0)

    def in_fire(slot, b):
        e0 = (chunkA + b * 8) * CHUNK
        pltpu.async_copy(src_hbm.at[pl.ds(e0, 1024)], isrc.at[slot], sem_in)
        pltpu.async_copy(dst_hbm.at[pl.ds(e0, 1024)], idst.at[slot], sem_in)
        pltpu.async_copy(val_hbm.at[pl.ds(e0, 1024)], ival.at[slot], sem_in)

    def in_wait(slot, b):
        e0 = (chunkA + b * 8) * CHUNK
        pltpu.make_async_copy(src_hbm.at[pl.ds(e0, 1024)], isrc.at[slot], sem_in).wait()
        pltpu.make_async_copy(dst_hbm.at[pl.ds(e0, 1024)], idst.at[slot], sem_in).wait()
        pltpu.make_async_copy(val_hbm.at[pl.ds(e0, 1024)], ival.at[slot], sem_in).wait()

    def compact_groups(slot, ngroups, cnt):
        for g in range(ngroups):
            sl = pl.ds(g * 16, 16)
            d = idst[slot, sl]
            sv = isrc[slot, sl]
            vv = ival[slot, sl]
            for h in range(2):
                m = (d < HALF) if h == 0 else (d >= HALF)
                local = d - h * HALF
                pos = plsc.cumsum(m.astype(jnp.int32))
                tgt = cnt[h] + pos - 1
                plsc.store_scatter(bsrc[h], [tgt], sv, mask=m)
                plsc.store_scatter(bdst[h], [tgt], local, mask=m)
                plsc.store_scatter(bval[h], [tgt], vv, mask=m)
                cnt[h] = cnt[h] + pos[15]
        return cnt

    def flush1024(h, f_h):
        base = (h * 32 + w) * SEG_CAP + f_h * CHUNK
        pltpu.sync_copy(bsrc[h].at[pl.ds(0, 1024)], osrc.at[pl.ds(base, 1024)])
        pltpu.sync_copy(bdst[h].at[pl.ds(0, 1024)], odst.at[pl.ds(base, 1024)])
        pltpu.sync_copy(bval[h].at[pl.ds(0, 1024)], oval.at[pl.ds(base, 1024)])

        def mv(i, _):
            sl_d = pl.ds(i * 16, 16)
            sl_s = pl.ds(1024 + i * 16, 16)
            bsrc[h][sl_d] = bsrc[h][sl_s]
            bdst[h][sl_d] = bdst[h][sl_s]
            bval[h][sl_d] = bval[h][sl_s]
            return 0
        lax.fori_loop(0, 64, mv, 0)

    def maybe_flush(cnt, fl):
        for h in range(2):
            full = cnt[h] >= 1024

            @pl.when(full)
            def _():
                flush1024(h, fl[h])
            cnt[h] = jnp.where(full, cnt[h] - 1024, cnt[h])
            fl[h] = jnp.where(full, fl[h] + 8, fl[h])
        return cnt, fl

    in_fire(0, 0)

    def dblk(di, carry):
        cnt = [carry[0], carry[1]]
        fl = [carry[2], carry[3]]
        for k in range(2):
            b = 2 * di + k
            in_wait(k, b)

            @pl.when(b < NBI - 1)
            def _():
                in_fire(1 - k, b + 1)
            cnt = compact_groups(k, 64, cnt)
            cnt, fl = maybe_flush(cnt, fl)
        return (cnt[0], cnt[1], fl[0], fl[1])

    z = jnp.int32(0)
    cnt0, cnt1, f0, f1 = lax.fori_loop(0, NBI // 2, dblk, (z, z, z, z))
    cnt = [cnt0, cnt1]
    fl = [f0, f1]

    def rem_chunk(r, carry):
        cnt = [carry[0], carry[1]]
        e0 = (chunkA + NBI * 8 + r) * CHUNK
        pltpu.sync_copy(src_hbm.at[pl.ds(e0, CHUNK)], isrc.at[0, pl.ds(0, CHUNK)])
        pltpu.sync_copy(dst_hbm.at[pl.ds(e0, CHUNK)], idst.at[0, pl.ds(0, CHUNK)])
        pltpu.sync_copy(val_hbm.at[pl.ds(e0, CHUNK)], ival.at[0, pl.ds(0, CHUNK)])
        cnt = compact_groups(0, 8, cnt)
        return (cnt[0], cnt[1])
    cnt0, cnt1 = lax.fori_loop(0, ncw - NBI * 8, rem_chunk, (cnt[0], cnt[1]))
    cnt = [cnt0, cnt1]
    cnt, fl = maybe_flush(cnt, fl)

    for h in range(2):
        ch = cnt[h]
        fh = fl[h]
        kfull = ch // CHUNK
        for i in range(8):
            @pl.when(i < kfull)
            def _():
                base = (h * 32 + w) * SEG_CAP + (fh + i) * CHUNK
                pltpu.sync_copy(bsrc[h].at[pl.ds(i * CHUNK, CHUNK)], osrc.at[pl.ds(base, CHUNK)])
                pltpu.sync_copy(bdst[h].at[pl.ds(i * CHUNK, CHUNK)], odst.at[pl.ds(base, CHUNK)])
                pltpu.sync_copy(bval[h].at[pl.ds(i * CHUNK, CHUNK)], oval.at[pl.ds(base, CHUNK)])
        fh = fh + kfull
        cr = ch % CHUNK
        npd = CHUNK - cr
        for i in range(8):
            @pl.when((i * 16 < npd) & (cr > 0))
            def _():
                sl = pl.ds(i * 16, 16)
                bsrc[h][pl.ds(ch + i * 16, 16)] = psrc[sl]
                bdst[h][pl.ds(ch + i * 16, 16)] = pdst[sl]
                bval[h][pl.ds(ch + i * 16, 16)] = pval[sl]

        @pl.when(cr > 0)
        def _():
            pbase = (h * 32 + w) * SEG_CAP + fh * CHUNK
            poff = kfull * CHUNK
            pltpu.sync_copy(bsrc[h].at[pl.ds(poff, CHUNK)], osrc.at[pl.ds(pbase, CHUNK)])
            pltpu.sync_copy(bdst[h].at[pl.ds(poff, CHUNK)], odst.at[pl.ds(pbase, CHUNK)])
            pltpu.sync_copy(bval[h].at[pl.ds(poff, CHUNK)], oval.at[pl.ds(pbase, CHUNK)])
        fh = fh + (cr > 0).astype(jnp.int32)
        target = jnp.maximum(((fh + 3) // 4) * 4, 4)
        for i in range(4):
            @pl.when(i < target - fh)
            def _():
                base = (h * 32 + w) * SEG_CAP + (fh + i) * CHUNK
                pltpu.sync_copy(psrc, osrc.at[pl.ds(base, CHUNK)])
                pltpu.sync_copy(pdst, odst.at[pl.ds(base, CHUNK)])
                pltpu.sync_copy(pval, oval.at[pl.ds(base, CHUNK)])
        cnt_v[pl.ds(0, 16)] = jnp.zeros((16,), jnp.int32) + target
        pltpu.sync_copy(cnt_v, ocnt.at[h * 32 + w])


@functools.partial(
    pl.kernel,
    out_type=[
        jax.ShapeDtypeStruct((N_NODES, EMB), jnp.float32),
        jax.ShapeDtypeStruct((N_NODES, EMB), jnp.bfloat16),
    ],
    mesh=_mesh,
    compiler_params=pltpu.CompilerParams(use_tc_tiling_on_sc=False,
                                         needs_layout_passes=False),
    scratch_types=[
        pltpu.VMEM((2, BLK * CHUNK), jnp.int32),
        pltpu.VMEM((2, BLK * CHUNK), jnp.int32),
        pltpu.VMEM((2, BLK * CHUNK), jnp.float32),
        pltpu.VMEM((2, BLK, CHUNK), jnp.int32),
        pltpu.VMEM((2, BLK, CHUNK, EMB), jnp.bfloat16),
        pltpu.VMEM((2, BLK, CHUNK, EMB), jnp.float32),
        pltpu.VMEM((16,), jnp.int32),
        pltpu.VMEM_SHARED((ACC_ROWS, EMB), jnp.float32),
        pltpu.SemaphoreType.DMA,
        pltpu.SemaphoreType.DMA,
        pltpu.SemaphoreType.DMA,
    ],
)
def _spmm_layer(emb_hbm, src_hbm, dst_hbm, val_hbm, cnt_hbm,
                out_hbm, obf_hbm,
                src_v, dst_v, val_v, sidx_v, rowsb_v, rows_v, cnt_v, acc,
                sem_i, sem_g, sem_s):
    c = lax.axis_index("c")
    s = lax.axis_index("s")
    base_node = c * HALF
    iota16 = lax.iota(jnp.int32, 16)

    def zz(i, _):
        rows_v[0, 0, i, pl.ds(0, 16)] = jnp.zeros((16,), jnp.float32)
        rows_v[0, 0, i, pl.ds(16, 16)] = jnp.zeros((16,), jnp.float32)
        return 0
    lax.fori_loop(0, CHUNK, zz, 0)
    row_base = s * (BASE_G * 8) + 8 * jnp.minimum(s, EXTRA_G_TILES)

    def zcopy(k, _):
        pltpu.sync_copy(rows_v.at[0, 0], acc.at[pl.ds(row_base + k * CHUNK, CHUNK)])
        return 0
    lax.fori_loop(0, 24, zcopy, 0)
    pltpu.sync_copy(rows_v.at[0, 0, pl.ds(0, 48)], acc.at[pl.ds(row_base + 3072, 48)])

    @pl.when(s < EXTRA_G_TILES)
    def _():
        pltpu.sync_copy(rows_v.at[0, 0, pl.ds(0, 8)], acc.at[pl.ds(row_base + 3120, 8)])
    plsc.subcore_barrier()

    def idx_fire(chunk0, slot, bidx):
        e0 = (chunk0 + bidx * BLK) * CHUNK
        ne = BLK * CHUNK
        pltpu.async_copy(src_hbm.at[pl.ds(e0, ne)], src_v.at[slot], sem_i)
        pltpu.async_copy(dst_hbm.at[pl.ds(e0, ne)], dst_v.at[slot], sem_i)
        pltpu.async_copy(val_hbm.at[pl.ds(e0, ne)], val_v.at[slot], sem_i)

    def idx_wait(chunk0, slot, bidx):
        e0 = (chunk0 + bidx * BLK) * CHUNK
        ne = BLK * CHUNK
        pltpu.make_async_copy(src_hbm.at[pl.ds(e0, ne)], src_v.at[slot], sem_i).wait()
        pltpu.make_async_copy(dst_hbm.at[pl.ds(e0, ne)], dst_v.at[slot], sem_i).wait()
        pltpu.make_async_copy(val_hbm.at[pl.ds(e0, ne)], val_v.at[slot], sem_i).wait()

    def gather_fire(slot):
        for j in range(BLK):
            pltpu.async_copy(
                emb_hbm.at[src_v.at[slot, pl.ds(j * CHUNK, CHUNK)]],
                rowsb_v.at[slot, j], sem_g)

    def gather_wait(slot):
        for j in range(BLK):
            pltpu.make_async_copy(
                emb_hbm.at[src_v.at[slot, pl.ds(j * CHUNK, CHUNK)]],
                rowsb_v.at[slot, j], sem_g).wait()

    def scatter_fire(slot):
        for j in range(BLK):
            pltpu.async_copy(rows_v.at[slot, j], acc.at[sidx_v.at[slot, j]],
                             sem_s, add=True)

    def scatter_wait(slot):
        for j in range(BLK):
            pltpu.make_async_copy(rows_v.at[slot, j], acc.at[sidx_v.at[slot, j]],
                                  sem_s).wait()

    def compute(slot):
        for j in range(BLK):
            for cc in range(8):
                off = cc * 16
                dstv = dst_v[slot, pl.ds(j * CHUNK + off, 16)]
                in_range = dstv < HALF
                dummy = HALF + j * CHUNK + off + iota16
                sidx_v[slot, j, pl.ds(off, 16)] = jnp.where(in_range, dstv, dummy)
                valv = val_v[slot, pl.ds(j * CHUNK + off, 16)]
                for e in range(16):
                    vs = jnp.broadcast_to(valv[e], (16,))
                    ce = off + e
                    lo, hi = plsc.unpack(rowsb_v[slot, j, ce, pl.ds(0, EMB)],
                                         format=plsc.PackFormat.INTERLEAVED)
                    rows_v[slot, j, ce, pl.ds(0, 16)] = lo * vs
                    rows_v[slot, j, ce, pl.ds(16, 16)] = hi * vs

    for k in range(2):
        r = c * 32 + 2 * s + k
        pltpu.sync_copy(cnt_hbm.at[r], cnt_v)
        nchunks = cnt_v[pl.ds(0, 16)][0]
        nd = nchunks // (2 * BLK)
        chunk0 = r * SEG_CHUNKS

        idx_fire(chunk0, 0, 0)
        idx_wait(chunk0, 0, 0)
        gather_fire(0)
        idx_fire(chunk0, 1, 1)

        def dbody(d, _):
            gather_wait(0)
            compute(0)

            @pl.when(d > 0)
            def _():
                scatter_wait(1)
            idx_wait(chunk0, 1, 2 * d + 1)
            gather_fire(1)
            scatter_fire(0)

            @pl.when(d < nd - 1)
            def _():
                idx_fire(chunk0, 0, 2 * d + 2)

            gather_wait(1)
            compute(1)
            scatter_wait(0)

            @pl.when(d < nd - 1)
            def _():
                idx_wait(chunk0, 0, 2 * d + 2)
                gather_fire(0)
            scatter_fire(1)

            @pl.when(d < nd - 1)
            def _():
                idx_fire(chunk0, 1, 2 * d + 3)
            return 0
        lax.fori_loop(0, nd, dbody, 0)
        scatter_wait(1)

    plsc.subcore_barrier()

    out_base = base_node + row_base
    pltpu.sync_copy(acc.at[pl.ds(row_base, 1024)], out_hbm.at[pl.ds(out_base, 1024)])
    pltpu.sync_copy(acc.at[pl.ds(row_base + 1024, 1024)], out_hbm.at[pl.ds(out_base + 1024, 1024)])
    pltpu.sync_copy(acc.at[pl.ds(row_base + 2048, 1024)], out_hbm.at[pl.ds(out_base + 2048, 1024)])
    pltpu.sync_copy(acc.at[pl.ds(row_base + 3072, 48)], out_hbm.at[pl.ds(out_base + 3072, 48)])

    @pl.when(s < EXTRA_G_TILES)
    def _():
        pltpu.sync_copy(acc.at[pl.ds(row_base + 3120, 8)], out_hbm.at[pl.ds(out_base + 3120, 8)])

    nrows = jnp.where(s < EXTRA_G_TILES, 3128, 3120)

    def bfchunk(k, _):
        roff = row_base + k * CHUNK
        pltpu.sync_copy(acc.at[pl.ds(roff, CHUNK)], rows_v.at[0, 0])

        def bfrow(r, _):
            a = rows_v[0, 0, r, pl.ds(0, 16)]
            b = rows_v[0, 0, r, pl.ds(16, 16)]
            rowsb_v[0, 0, r, pl.ds(0, EMB)] = plsc.pack(
                a, b, format=plsc.PackFormat.INTERLEAVED)
            return 0
        lax.fori_loop(0, CHUNK, bfrow, 0)
        pltpu.sync_copy(rowsb_v.at[0, 0], obf_hbm.at[pl.ds(out_base + k * CHUNK, CHUNK)])
        return 0
    lax.fori_loop(0, nrows // CHUNK, bfchunk, 0)

    tail0 = (nrows // CHUNK) * CHUNK
    ntail = nrows - tail0
    pltpu.sync_copy(acc.at[pl.ds(row_base + tail0, 48)], rows_v.at[0, 0, pl.ds(0, 48)])

    @pl.when(s < EXTRA_G_TILES)
    def _():
        pltpu.sync_copy(acc.at[pl.ds(row_base + tail0 + 48, 8)],
                        rows_v.at[0, 0, pl.ds(48, 8)])

    def bftail(r, _):
        a = rows_v[0, 0, r, pl.ds(0, 16)]
        b = rows_v[0, 0, r, pl.ds(16, 16)]
        rowsb_v[0, 0, r, pl.ds(0, EMB)] = plsc.pack(
            a, b, format=plsc.PackFormat.INTERLEAVED)
        return 0
    lax.fori_loop(0, ntail, bftail, 0)
    pltpu.sync_copy(rowsb_v.at[0, 0, pl.ds(0, 48)],
                    obf_hbm.at[pl.ds(out_base + tail0, 48)])

    @pl.when(s < EXTRA_G_TILES)
    def _():
        pltpu.sync_copy(rowsb_v.at[0, 0, pl.ds(48, 8)],
                        obf_hbm.at[pl.ds(out_base + tail0 + 48, 8)])


@functools.partial(
    pl.kernel,
    out_type=jax.ShapeDtypeStruct((8192, EMB), jnp.float32),
    mesh=_mesh,
    compiler_params=pltpu.CompilerParams(use_tc_tiling_on_sc=False),
    scratch_types=[
        pltpu.VMEM((256,), jnp.int32),
        pltpu.VMEM((2, CHUNK, EMB), jnp.float32),
        pltpu.VMEM((2, CHUNK, EMB), jnp.float32),
        pltpu.VMEM((2, CHUNK, EMB), jnp.float32),
        pltpu.SemaphoreType.DMA,
    ],
)
def _gather_mean(e0_hbm, e1_hbm, e2_hbm, ids_hbm, out_hbm,
                 idx_v, a_v, b_v, c_v, sem):
    c = lax.axis_index("c")
    s = lax.axis_index("s")
    wid = s * 2 + c
    pltpu.sync_copy(ids_hbm.at[pl.ds(wid * 256, 256)], idx_v)
    handles = []
    for j in range(2):
        isl = idx_v.at[pl.ds(j * CHUNK, CHUNK)]
        handles.append(pltpu.async_copy(e0_hbm.at[isl], a_v.at[j], sem))
        handles.append(pltpu.async_copy(e1_hbm.at[isl], b_v.at[j], sem))
        handles.append(pltpu.async_copy(e2_hbm.at[isl], c_v.at[j], sem))
    for h in handles:
        h.wait()
    third = jnp.full((16,), 1.0 / 3.0, jnp.float32)
    for j in range(2):
        def mean_row(r, _):
            for hh in range(2):
                sl = pl.ds(hh * 16, 16)
                a_v[j, r, sl] = (a_v[j, r, sl] + b_v[j, r, sl] + c_v[j, r, sl]) * third
            return 0
        lax.fori_loop(0, CHUNK, mean_row, 0)
        pltpu.sync_copy(a_v.at[j], out_hbm.at[pl.ds(wid * 256 + j * CHUNK, CHUNK)])


def _mlp_body(uf0, uf1, uf2, uf3, if0, if1, if2, if3, ug, ig,
              wu0, wu1, wu2, wu3, wi0, wi1, wi2, wi3,
              fc1w, fc1b, fc2w, fc2b, outw, out_ref):
    def dot_t(a, b):
        return lax.dot_general(a, b, (((1,), (1,)), ((), ())),
                               preferred_element_type=jnp.float32)

    h1 = dot_t(ug[...], fc1w[:, 128:160])
    h1 = h1 + dot_t(ig[...], fc1w[:, 288:320])
    ufs = (uf0, uf1, uf2, uf3)
    wus = (wu0, wu1, wu2, wu3)
    ifs = (if0, if1, if2, if3)
    wis = (wi0, wi1, wi2, wi3)
    for f in range(4):
        tu = dot_t(ufs[f][...], wus[f][...])
        h1 = h1 + dot_t(tu, fc1w[:, 32 * f:32 * f + 32])
        ti = dot_t(ifs[f][...], wis[f][...])
        h1 = h1 + dot_t(ti, fc1w[:, 160 + 32 * f:160 + 32 * f + 32])
    h1 = jnp.maximum(h1 + fc1b[...], 0.0)
    h2 = jnp.maximum(dot_t(h1, fc2w[...]) + fc2b[...], 0.0)
    out_ref[...] = dot_t(h2, outw[...])


def kernel(user_feat_0, user_feat_1, user_feat_2, user_feat_3,
           item_feat_0, item_feat_1, item_feat_2, item_feat_3,
           user_ids, item_ids, adj_indices, adj_values,
           user_emb, item_emb,
           Wu0, Wu1, Wu2, Wu3, Wi0, Wi1, Wi2, Wi3,
           fc1_w, fc1_b, fc2_w, fc2_b, out_w, out_b):
    src1d = adj_indices[0].reshape(N_EDGES)
    dst1d = adj_indices[1].reshape(N_EDGES)
    val1d = adj_values.reshape(N_EDGES)

    seg_src, seg_dst, seg_val, seg_cnt = _partition_edges(src1d, dst1d, val1d)

    emb0 = jnp.concatenate([user_emb, item_emb], axis=0)
    e0b = emb0.astype(jnp.bfloat16)
    emb0_bf = jnp.stack([e0b[:, :16], e0b[:, 16:]], axis=-1).reshape(N_NODES, EMB)
    emb1, emb1_bf = _spmm_layer(emb0_bf, seg_src, seg_dst, seg_val, seg_cnt)
    emb2, _ = _spmm_layer(emb1_bf, seg_src, seg_dst, seg_val, seg_cnt)

    ids1d = jnp.concatenate([user_ids, item_ids + NUM_USERS]).astype(jnp.int32)
    gcn = _gather_mean(emb0, emb1, emb2, ids1d)
    ugcn = gcn[:4096]
    igcn = gcn[4096:]

    bs = 512
    grid = (4096 // bs,)
    feat_spec = pl.BlockSpec((bs, 512), lambda i: (i, 0))
    gcn_spec = pl.BlockSpec((bs, EMB), lambda i: (i, 0))
    w_spec = pl.BlockSpec((EMB, 512), lambda i: (0, 0))
    out = pl.pallas_call(
        _mlp_body,
        grid=grid,
        in_specs=[feat_spec] * 4 + [feat_spec] * 4 + [gcn_spec] * 2
        + [w_spec] * 8
        + [pl.BlockSpec((128, 320), lambda i: (0, 0)),
           pl.BlockSpec((1, 128), lambda i: (0, 0)),
           pl.BlockSpec((64, 128), lambda i: (0, 0)),
           pl.BlockSpec((1, 64), lambda i: (0, 0)),
           pl.BlockSpec((1, 64), lambda i: (0, 0))],
        out_specs=pl.BlockSpec((bs, 1), lambda i: (i, 0)),
        out_shape=jax.ShapeDtypeStruct((4096, 1), jnp.float32),
    )(user_feat_0, user_feat_1, user_feat_2, user_feat_3,
      item_feat_0, item_feat_1, item_feat_2, item_feat_3,
      ugcn, igcn,
      Wu0, Wu1, Wu2, Wu3, Wi0, Wi1, Wi2, Wi3,
      fc1_w, fc1_b.reshape(1, 128), fc2_w, fc2_b.reshape(1, 64),
      out_w)
    return out + out_b


# speedup vs baseline: 20.6299x; 1.0826x over previous
import functools

import jax
import jax.numpy as jnp
from jax import lax
from jax.experimental import pallas as pl
from jax.experimental.pallas import tpu as pltpu
from jax.experimental.pallas import tpu_sc as plsc

NUM_USERS = 50000
NUM_ITEMS = 50000
N_NODES = NUM_USERS + NUM_ITEMS
HALF = N_NODES // 2
EMB = 32
N_EDGES = 1600000
CHUNK = 128
N_CHUNKS = N_EDGES // CHUNK
BLK = 2
DUMMY_ROWS = BLK * CHUNK
ACC_ROWS = HALF + DUMMY_ROWS
SEG_CHUNKS = 396
SEG_CAP = SEG_CHUNKS * CHUNK
N_SEGS = 64
P_BASE_CHUNKS = N_CHUNKS // 32
P_EXTRA = N_CHUNKS - 32 * P_BASE_CHUNKS
BASE_G = 390
EXTRA_G_TILES = 10

_mesh = plsc.VectorSubcoreMesh(core_axis_name="c", subcore_axis_name="s")


@functools.partial(
    pl.kernel,
    out_type=[
        jax.ShapeDtypeStruct((N_SEGS * SEG_CAP,), jnp.int32),
        jax.ShapeDtypeStruct((N_SEGS * SEG_CAP,), jnp.int32),
        jax.ShapeDtypeStruct((N_SEGS * SEG_CAP,), jnp.float32),
        jax.ShapeDtypeStruct((N_SEGS, 16), jnp.int32),
    ],
    mesh=_mesh,
    compiler_params=pltpu.CompilerParams(use_tc_tiling_on_sc=False,
                                         needs_layout_passes=False),
    scratch_types=[
        pltpu.VMEM((2, 1024), jnp.int32),
        pltpu.VMEM((2, 1024), jnp.int32),
        pltpu.VMEM((2, 1024), jnp.float32),
        pltpu.VMEM((2304,), jnp.int32),
        pltpu.VMEM((2304,), jnp.int32),
        pltpu.VMEM((2304,), jnp.int32),
        pltpu.VMEM((2304,), jnp.int32),
        pltpu.VMEM((2304,), jnp.float32),
        pltpu.VMEM((2304,), jnp.float32),
        pltpu.VMEM((CHUNK,), jnp.int32),
        pltpu.VMEM((CHUNK,), jnp.int32),
        pltpu.VMEM((CHUNK,), jnp.float32),
        pltpu.VMEM((16,), jnp.int32),
        pltpu.SemaphoreType.DMA,
    ],
)
def _partition_edges(src_hbm, dst_hbm, val_hbm,
                     osrc, odst, oval, ocnt,
                     isrc, idst, ival, bsrc0, bsrc1, bdst0, bdst1,
                     bval0, bval1, psrc, pdst, pval, cnt_v, sem_in):
    bsrc = (bsrc0, bsrc1)
    bdst = (bdst0, bdst1)
    bval = (bval0, bval1)
    c = lax.axis_index("c")
    s = lax.axis_index("s")
    w = s * 2 + c
    iota16 = lax.iota(jnp.int32, 16)
    chunkA = w * P_BASE_CHUNKS + jnp.minimum(w, P_EXTRA)
    ncw = jnp.where(w < P_EXTRA, P_BASE_CHUNKS + 1, P_BASE_CHUNKS)
    NBI = 48

    def fillpad(i, _):
        sl = pl.ds(i * 16, 16)
        psrc[sl] = w * CHUNK + i * 16 + iota16
        pdst[sl] = jnp.full((16,), HALF, jnp.int32)
        pval[sl] = jnp.zeros((16,), jnp.float32)
        return 0
    lax.fori_loop(0, 8, fillpad, 0)

    def in_fire(slot, b):
        e0 = (chunkA + b * 8) * CHUNK
        pltpu.async_copy(src_hbm.at[pl.ds(e0, 1024)], isrc.at[slot], sem_in)
        pltpu.async_copy(dst_hbm.at[pl.ds(e0, 1024)], idst.at[slot], sem_in)
        pltpu.async_copy(val_hbm.at[pl.ds(e0, 1024)], ival.at[slot], sem_in)

    def in_wait(slot, b):
        e0 = (chunkA + b * 8) * CHUNK
        pltpu.make_async_copy(src_hbm.at[pl.ds(e0, 1024)], isrc.at[slot], sem_in).wait()
        pltpu.make_async_copy(dst_hbm.at[pl.ds(e0, 1024)], idst.at[slot], sem_in).wait()
        pltpu.make_async_copy(val_hbm.at[pl.ds(e0, 1024)], ival.at[slot], sem_in).wait()

    def compact_groups(slot, ngroups, cnt):
        for g in range(ngroups):
            sl = pl.ds(g * 16, 16)
            d = idst[slot, sl]
            sv = isrc[slot, sl]
            vv = ival[slot, sl]
            for h in range(2):
                m = (d < HALF) if h == 0 else (d >= HALF)
                local = d - h * HALF
                pos = plsc.cumsum(m.astype(jnp.int32))
                tgt = cnt[h] + pos - 1
                plsc.store_scatter(bsrc[h], [tgt], sv, mask=m)
                plsc.store_scatter(bdst[h], [tgt], local, mask=m)
                plsc.store_scatter(bval[h], [tgt], vv, mask=m)
                cnt[h] = cnt[h] + pos[15]
        return cnt

    def flush1024(h, f_h):
        base = (h * 32 + w) * SEG_CAP + f_h * CHUNK
        pltpu.sync_copy(bsrc[h].at[pl.ds(0, 1024)], osrc.at[pl.ds(base, 1024)])
        pltpu.sync_copy(bdst[h].at[pl.ds(0, 1024)], odst.at[pl.ds(base, 1024)])
        pltpu.sync_copy(bval[h].at[pl.ds(0, 1024)], oval.at[pl.ds(base, 1024)])

        def mv(i, _):
            sl_d = pl.ds(i * 16, 16)
            sl_s = pl.ds(1024 + i * 16, 16)
            bsrc[h][sl_d] = bsrc[h][sl_s]
            bdst[h][sl_d] = bdst[h][sl_s]
            bval[h][sl_d] = bval[h][sl_s]
            return 0
        lax.fori_loop(0, 64, mv, 0)

    def maybe_flush(cnt, fl):
        for h in range(2):
            full = cnt[h] >= 1024

            @pl.when(full)
            def _():
                flush1024(h, fl[h])
            cnt[h] = jnp.where(full, cnt[h] - 1024, cnt[h])
            fl[h] = jnp.where(full, fl[h] + 8, fl[h])
        return cnt, fl

    in_fire(0, 0)

    def dblk(di, carry):
        cnt = [carry[0], carry[1]]
        fl = [carry[2], carry[3]]
        for k in range(2):
            b = 2 * di + k
            in_wait(k, b)

            @pl.when(b < NBI - 1)
            def _():
                in_fire(1 - k, b + 1)
            cnt = compact_groups(k, 64, cnt)
            cnt, fl = maybe_flush(cnt, fl)
        return (cnt[0], cnt[1], fl[0], fl[1])

    z = jnp.int32(0)
    cnt0, cnt1, f0, f1 = lax.fori_loop(0, NBI // 2, dblk, (z, z, z, z))
    cnt = [cnt0, cnt1]
    fl = [f0, f1]

    def rem_chunk(r, carry):
        cnt = [carry[0], carry[1]]
        e0 = (chunkA + NBI * 8 + r) * CHUNK
        pltpu.sync_copy(src_hbm.at[pl.ds(e0, CHUNK)], isrc.at[0, pl.ds(0, CHUNK)])
        pltpu.sync_copy(dst_hbm.at[pl.ds(e0, CHUNK)], idst.at[0, pl.ds(0, CHUNK)])
        pltpu.sync_copy(val_hbm.at[pl.ds(e0, CHUNK)], ival.at[0, pl.ds(0, CHUNK)])
        cnt = compact_groups(0, 8, cnt)
        return (cnt[0], cnt[1])
    cnt0, cnt1 = lax.fori_loop(0, ncw - NBI * 8, rem_chunk, (cnt[0], cnt[1]))
    cnt = [cnt0, cnt1]
    cnt, fl = maybe_flush(cnt, fl)

    for h in range(2):
        ch = cnt[h]
        fh = fl[h]
        kfull = ch // CHUNK
        for i in range(8):
            @pl.when(i < kfull)
            def _():
                base = (h * 32 + w) * SEG_CAP + (fh + i) * CHUNK
                pltpu.sync_copy(bsrc[h].at[pl.ds(i * CHUNK, CHUNK)], osrc.at[pl.ds(base, CHUNK)])
                pltpu.sync_copy(bdst[h].at[pl.ds(i * CHUNK, CHUNK)], odst.at[pl.ds(base, CHUNK)])
                pltpu.sync_copy(bval[h].at[pl.ds(i * CHUNK, CHUNK)], oval.at[pl.ds(base, CHUNK)])
        fh = fh + kfull
        cr = ch % CHUNK
        npd = CHUNK - cr
        for i in range(8):
            @pl.when((i * 16 < npd) & (cr > 0))
            def _():
                sl = pl.ds(i * 16, 16)
                bsrc[h][pl.ds(ch + i * 16, 16)] = psrc[sl]
                bdst[h][pl.ds(ch + i * 16, 16)] = pdst[sl]
                bval[h][pl.ds(ch + i * 16, 16)] = pval[sl]

        @pl.when(cr > 0)
        def _():
            pbase = (h * 32 + w) * SEG_CAP + fh * CHUNK
            poff = kfull * CHUNK
            pltpu.sync_copy(bsrc[h].at[pl.ds(poff, CHUNK)], osrc.at[pl.ds(pbase, CHUNK)])
            pltpu.sync_copy(bdst[h].at[pl.ds(poff, CHUNK)], odst.at[pl.ds(pbase, CHUNK)])
            pltpu.sync_copy(bval[h].at[pl.ds(poff, CHUNK)], oval.at[pl.ds(pbase, CHUNK)])
        fh = fh + (cr > 0).astype(jnp.int32)
        target = jnp.maximum(((fh + 3) // 4) * 4, 4)
        for i in range(4):
            @pl.when(i < target - fh)
            def _():
                base = (h * 32 + w) * SEG_CAP + (fh + i) * CHUNK
                pltpu.sync_copy(psrc, osrc.at[pl.ds(base, CHUNK)])
                pltpu.sync_copy(pdst, odst.at[pl.ds(base, CHUNK)])
                pltpu.sync_copy(pval, oval.at[pl.ds(base, CHUNK)])
        cnt_v[pl.ds(0, 16)] = jnp.zeros((16,), jnp.int32) + target
        pltpu.sync_copy(cnt_v, ocnt.at[h * 32 + w])


@functools.partial(
    pl.kernel,
    out_type=jax.ShapeDtypeStruct((N_NODES, EMB), jnp.float32),
    mesh=_mesh,
    compiler_params=pltpu.CompilerParams(use_tc_tiling_on_sc=False),
    scratch_types=[
        pltpu.VMEM((2, BLK * CHUNK), jnp.int32),
        pltpu.VMEM((2, BLK * CHUNK), jnp.int32),
        pltpu.VMEM((2, BLK * CHUNK), jnp.float32),
        pltpu.VMEM((2, BLK, CHUNK), jnp.int32),
        pltpu.VMEM((2, BLK, CHUNK, EMB), jnp.float32),
        pltpu.VMEM((16,), jnp.int32),
        pltpu.VMEM_SHARED((ACC_ROWS, EMB), jnp.float32),
        pltpu.SemaphoreType.DMA,
        pltpu.SemaphoreType.DMA,
        pltpu.SemaphoreType.DMA,
    ],
)
def _spmm_layer(emb_hbm, src_hbm, dst_hbm, val_hbm, cnt_hbm, out_hbm,
                src_v, dst_v, val_v, sidx_v, rows_v, cnt_v, acc,
                sem_i, sem_g, sem_s):
    c = lax.axis_index("c")
    s = lax.axis_index("s")
    base_node = c * HALF
    iota16 = lax.iota(jnp.int32, 16)

    def zz(i, _):
        rows_v[0, 0, i, pl.ds(0, 16)] = jnp.zeros((16,), jnp.float32)
        rows_v[0, 0, i, pl.ds(16, 16)] = jnp.zeros((16,), jnp.float32)
        return 0
    lax.fori_loop(0, CHUNK, zz, 0)
    row_base = s * (BASE_G * 8) + 8 * jnp.minimum(s, EXTRA_G_TILES)

    def zcopy(k, _):
        pltpu.sync_copy(rows_v.at[0, 0], acc.at[pl.ds(row_base + k * CHUNK, CHUNK)])
        return 0
    lax.fori_loop(0, 24, zcopy, 0)
    pltpu.sync_copy(rows_v.at[0, 0, pl.ds(0, 48)], acc.at[pl.ds(row_base + 3072, 48)])

    @pl.when(s < EXTRA_G_TILES)
    def _():
        pltpu.sync_copy(rows_v.at[0, 0, pl.ds(0, 8)], acc.at[pl.ds(row_base + 3120, 8)])
    plsc.subcore_barrier()

    def idx_fire(chunk0, slot, bidx):
        e0 = (chunk0 + bidx * BLK) * CHUNK
        ne = BLK * CHUNK
        pltpu.async_copy(src_hbm.at[pl.ds(e0, ne)], src_v.at[slot], sem_i)
        pltpu.async_copy(dst_hbm.at[pl.ds(e0, ne)], dst_v.at[slot], sem_i)
        pltpu.async_copy(val_hbm.at[pl.ds(e0, ne)], val_v.at[slot], sem_i)

    def idx_wait(chunk0, slot, bidx):
        e0 = (chunk0 + bidx * BLK) * CHUNK
        ne = BLK * CHUNK
        pltpu.make_async_copy(src_hbm.at[pl.ds(e0, ne)], src_v.at[slot], sem_i).wait()
        pltpu.make_async_copy(dst_hbm.at[pl.ds(e0, ne)], dst_v.at[slot], sem_i).wait()
        pltpu.make_async_copy(val_hbm.at[pl.ds(e0, ne)], val_v.at[slot], sem_i).wait()

    def gather_fire(slot):
        for j in range(BLK):
            pltpu.async_copy(
                emb_hbm.at[src_v.at[slot, pl.ds(j * CHUNK, CHUNK)]],
                rows_v.at[slot, j], sem_g)

    def gather_wait(slot):
        for j in range(BLK):
            pltpu.make_async_copy(
                emb_hbm.at[src_v.at[slot, pl.ds(j * CHUNK, CHUNK)]],
                rows_v.at[slot, j], sem_g).wait()

    def scatter_fire(slot):
        for j in range(BLK):
            pltpu.async_copy(rows_v.at[slot, j], acc.at[sidx_v.at[slot, j]],
                             sem_s, add=True)

    def scatter_wait(slot):
        for j in range(BLK):
            pltpu.make_async_copy(rows_v.at[slot, j], acc.at[sidx_v.at[slot, j]],
                                  sem_s).wait()

    def compute(slot):
        for j in range(BLK):
            for cc in range(8):
                off = cc * 16
                dstv = dst_v[slot, pl.ds(j * CHUNK + off, 16)]
                in_range = dstv < HALF
                dummy = HALF + j * CHUNK + off + iota16
                sidx_v[slot, j, pl.ds(off, 16)] = jnp.where(in_range, dstv, dummy)
                valv = val_v[slot, pl.ds(j * CHUNK + off, 16)]
                for e in range(16):
                    vs = jnp.broadcast_to(valv[e], (16,))
                    ce = off + e
                    rows_v[slot, j, ce, pl.ds(0, 16)] = rows_v[slot, j, ce, pl.ds(0, 16)] * vs
                    rows_v[slot, j, ce, pl.ds(16, 16)] = rows_v[slot, j, ce, pl.ds(16, 16)] * vs

    for k in range(2):
        r = c * 32 + 2 * s + k
        pltpu.sync_copy(cnt_hbm.at[r], cnt_v)
        nchunks = cnt_v[pl.ds(0, 16)][0]
        nd = nchunks // (2 * BLK)
        chunk0 = r * SEG_CHUNKS

        idx_fire(chunk0, 0, 0)
        idx_wait(chunk0, 0, 0)
        gather_fire(0)
        idx_fire(chunk0, 1, 1)

        def dbody(d, _):
            gather_wait(0)
            compute(0)

            @pl.when(d > 0)
            def _():
                scatter_wait(1)
            idx_wait(chunk0, 1, 2 * d + 1)
            gather_fire(1)
            scatter_fire(0)

            @pl.when(d < nd - 1)
            def _():
                idx_fire(chunk0, 0, 2 * d + 2)

            gather_wait(1)
            compute(1)
            scatter_wait(0)

            @pl.when(d < nd - 1)
            def _():
                idx_wait(chunk0, 0, 2 * d + 2)
                gather_fire(0)
            scatter_fire(1)

            @pl.when(d < nd - 1)
            def _():
                idx_fire(chunk0, 1, 2 * d + 3)
            return 0
        lax.fori_loop(0, nd, dbody, 0)
        scatter_wait(1)

    plsc.subcore_barrier()

    out_base = base_node + row_base
    pltpu.sync_copy(acc.at[pl.ds(row_base, 1024)], out_hbm.at[pl.ds(out_base, 1024)])
    pltpu.sync_copy(acc.at[pl.ds(row_base + 1024, 1024)], out_hbm.at[pl.ds(out_base + 1024, 1024)])
    pltpu.sync_copy(acc.at[pl.ds(row_base + 2048, 1024)], out_hbm.at[pl.ds(out_base + 2048, 1024)])
    pltpu.sync_copy(acc.at[pl.ds(row_base + 3072, 48)], out_hbm.at[pl.ds(out_base + 3072, 48)])

    @pl.when(s < EXTRA_G_TILES)
    def _():
        pltpu.sync_copy(acc.at[pl.ds(row_base + 3120, 8)], out_hbm.at[pl.ds(out_base + 3120, 8)])


@functools.partial(
    pl.kernel,
    out_type=jax.ShapeDtypeStruct((8192, EMB), jnp.float32),
    mesh=_mesh,
    compiler_params=pltpu.CompilerParams(use_tc_tiling_on_sc=False),
    scratch_types=[
        pltpu.VMEM((256,), jnp.int32),
        pltpu.VMEM((2, CHUNK, EMB), jnp.float32),
        pltpu.VMEM((2, CHUNK, EMB), jnp.float32),
        pltpu.VMEM((2, CHUNK, EMB), jnp.float32),
        pltpu.SemaphoreType.DMA,
    ],
)
def _gather_mean(e0_hbm, e1_hbm, e2_hbm, ids_hbm, out_hbm,
                 idx_v, a_v, b_v, c_v, sem):
    c = lax.axis_index("c")
    s = lax.axis_index("s")
    wid = s * 2 + c
    pltpu.sync_copy(ids_hbm.at[pl.ds(wid * 256, 256)], idx_v)
    handles = []
    for j in range(2):
        isl = idx_v.at[pl.ds(j * CHUNK, CHUNK)]
        handles.append(pltpu.async_copy(e0_hbm.at[isl], a_v.at[j], sem))
        handles.append(pltpu.async_copy(e1_hbm.at[isl], b_v.at[j], sem))
        handles.append(pltpu.async_copy(e2_hbm.at[isl], c_v.at[j], sem))
    for h in handles:
        h.wait()
    third = jnp.full((16,), 1.0 / 3.0, jnp.float32)
    for j in range(2):
        def mean_row(r, _):
            for hh in range(2):
                sl = pl.ds(hh * 16, 16)
                a_v[j, r, sl] = (a_v[j, r, sl] + b_v[j, r, sl] + c_v[j, r, sl]) * third
            return 0
        lax.fori_loop(0, CHUNK, mean_row, 0)
        pltpu.sync_copy(a_v.at[j], out_hbm.at[pl.ds(wid * 256 + j * CHUNK, CHUNK)])


def _mlp_body(uf0, uf1, uf2, uf3, if0, if1, if2, if3, ug, ig,
              wu0, wu1, wu2, wu3, wi0, wi1, wi2, wi3,
              fc1w, fc1b, fc2w, fc2b, outw, out_ref):
    def dot_t(a, b):
        return lax.dot_general(a, b, (((1,), (1,)), ((), ())),
                               preferred_element_type=jnp.float32)

    h1 = dot_t(ug[...], fc1w[:, 128:160])
    h1 = h1 + dot_t(ig[...], fc1w[:, 288:320])
    ufs = (uf0, uf1, uf2, uf3)
    wus = (wu0, wu1, wu2, wu3)
    ifs = (if0, if1, if2, if3)
    wis = (wi0, wi1, wi2, wi3)
    for f in range(4):
        tu = dot_t(ufs[f][...], wus[f][...])
        h1 = h1 + dot_t(tu, fc1w[:, 32 * f:32 * f + 32])
        ti = dot_t(ifs[f][...], wis[f][...])
        h1 = h1 + dot_t(ti, fc1w[:, 160 + 32 * f:160 + 32 * f + 32])
    h1 = jnp.maximum(h1 + fc1b[...], 0.0)
    h2 = jnp.maximum(dot_t(h1, fc2w[...]) + fc2b[...], 0.0)
    out_ref[...] = dot_t(h2, outw[...])


def kernel(user_feat_0, user_feat_1, user_feat_2, user_feat_3,
           item_feat_0, item_feat_1, item_feat_2, item_feat_3,
           user_ids, item_ids, adj_indices, adj_values,
           user_emb, item_emb,
           Wu0, Wu1, Wu2, Wu3, Wi0, Wi1, Wi2, Wi3,
           fc1_w, fc1_b, fc2_w, fc2_b, out_w, out_b):
    src1d = adj_indices[0].reshape(N_EDGES)
    dst1d = adj_indices[1].reshape(N_EDGES)
    val1d = adj_values.reshape(N_EDGES)

    seg_src, seg_dst, seg_val, seg_cnt = _partition_edges(src1d, dst1d, val1d)

    emb0 = jnp.concatenate([user_emb, item_emb], axis=0)
    emb1 = _spmm_layer(emb0, seg_src, seg_dst, seg_val, seg_cnt)
    emb2 = _spmm_layer(emb1, seg_src, seg_dst, seg_val, seg_cnt)

    ids1d = jnp.concatenate([user_ids, item_ids + NUM_USERS]).astype(jnp.int32)
    gcn = _gather_mean(emb0, emb1, emb2, ids1d)
    ugcn = gcn[:4096]
    igcn = gcn[4096:]

    bs = 512
    grid = (4096 // bs,)
    feat_spec = pl.BlockSpec((bs, 512), lambda i: (i, 0))
    gcn_spec = pl.BlockSpec((bs, EMB), lambda i: (i, 0))
    w_spec = pl.BlockSpec((EMB, 512), lambda i: (0, 0))
    out = pl.pallas_call(
        _mlp_body,
        grid=grid,
        in_specs=[feat_spec] * 4 + [feat_spec] * 4 + [gcn_spec] * 2
        + [w_spec] * 8
        + [pl.BlockSpec((128, 320), lambda i: (0, 0)),
           pl.BlockSpec((1, 128), lambda i: (0, 0)),
           pl.BlockSpec((64, 128), lambda i: (0, 0)),
           pl.BlockSpec((1, 64), lambda i: (0, 0)),
           pl.BlockSpec((1, 64), lambda i: (0, 0))],
        out_specs=pl.BlockSpec((bs, 1), lambda i: (i, 0)),
        out_shape=jax.ShapeDtypeStruct((4096, 1), jnp.float32),
    )(user_feat_0, user_feat_1, user_feat_2, user_feat_3,
      item_feat_0, item_feat_1, item_feat_2, item_feat_3,
      ugcn, igcn,
      Wu0, Wu1, Wu2, Wu3, Wi0, Wi1, Wi2, Wi3,
      fc1_w, fc1_b.reshape(1, 128), fc2_w, fc2_b.reshape(1, 64),
      out_w)
    return out + out_b
